# Initial kernel scaffold; baseline (speedup 1.0000x reference)
import jax
import jax.numpy as jnp
from jax.experimental import pallas as pl


def kernel(scores):
    raise NotImplementedError("write your pallas kernel here")


# speedup vs baseline: 4.1872x; 4.1872x over previous
import functools

import jax
import jax.numpy as jnp
from jax import lax
from jax.experimental import pallas as pl

NUCLEUS_P = 0.9
TEMPERATURE = 1.0


def _roll(x, s, axis):
    n = x.shape[axis]
    s = s % n
    if s == 0:
        return x
    if axis == 0:
        return lax.concatenate([x[n - s:, :], x[: n - s, :]], 0)
    return lax.concatenate([x[:, n - s:], x[:, : n - s]], 1)


def _cumsum_last(x, log2n):
    n = x.shape[-1]
    for t in range(log2n):
        sh = 1 << t
        shifted = lax.concatenate(
            [jnp.zeros(x.shape[:-1] + (sh,), x.dtype), x[..., : n - sh]], x.ndim - 1)
        x = x + shifted
    return x


def _cumsum_ax0(x, log2n):
    n = x.shape[0]
    for t in range(log2n):
        sh = 1 << t
        shifted = lax.concatenate(
            [jnp.zeros((sh,) + x.shape[1:], x.dtype), x[: n - sh]], 0)
        x = x + shifted
    return x


@functools.lru_cache(maxsize=None)
def _build(batch, nvalid, log_r, log_c):
    R, C = 1 << log_r, 1 << log_c
    N = R * C
    LC, LN = log_c, log_r + log_c

    def body(s_ref, g_ref, np_ref, sel_ref):
        x = s_ref[0]
        iota0 = lax.broadcasted_iota(jnp.int32, (R, C), 0)
        iota1 = lax.broadcasted_iota(jnp.int32, (R, C), 1)

        def bit_of_i(bit):
            if bit >= LC:
                return (iota0 & (1 << (bit - LC))) != 0
            return (iota1 & (1 << bit)) != 0

        def compare_exchange(x, K, D):
            if D >= LC:
                axis, S = 0, 1 << (D - LC)
            else:
                axis, S = 1, 1 << D
            bD = bit_of_i(D)
            partner = jnp.where(bD, _roll(x, S, axis), _roll(x, -S, axis))
            mn = jnp.minimum(x, partner)
            mx = jnp.maximum(x, partner)
            return jnp.where(bD == bit_of_i(K), mx, mn)

        for K in range(1, LN + 1):
            for D in range(K - 1, -1, -1):
                x = compare_exchange(x, K, D)

        m = x[0, 0]
        e = jnp.exp((x - m) / TEMPERATURE)
        Z = jnp.sum(e)
        rz = 1.0 / Z
        q = e * rz

        cs = _cumsum_last(q, LC)
        row_tot = cs[:, C - 1:C]
        row_off = _cumsum_ax0(row_tot, log_r) - row_tot
        cum = cs + row_off

        iif = iota0.astype(jnp.float32) * float(C) + iota1.astype(jnp.float32)
        mask = (cum <= NUCLEUS_P) | (iif == 0.0)
        qn = jnp.where(mask, q, 0.0)
        Znuc = jnp.sum(qn)
        npv = qn / Znuc
        np_ref[0] = npv

        g = g_ref[0]
        t = jnp.where(mask, g + jnp.log(jnp.maximum(npv, 1e-30)), -jnp.inf)
        tmax = jnp.max(t)
        jstar = jnp.min(jnp.where(t == tmax, iif, float(N)))
        qstar = jnp.max(jnp.where(iif == jstar, q, -1.0))

        so = s_ref[0]
        eo = jnp.exp((so - m) / TEMPERATURE)
        qo = eo * rz
        cgt = jnp.sum(jnp.where(qo > qstar, 1.0, 0.0))
        eqf = jnp.where(qo == qstar, 1.0, 0.0)
        ics = _cumsum_last(eqf, LC)
        erow_tot = ics[:, C - 1:C]
        erow_off = _cumsum_ax0(erow_tot, log_r) - erow_tot
        prefix_excl = ics + erow_off - eqf
        kk = jstar - cgt
        sel = jnp.min(jnp.where((eqf > 0.0) & (prefix_excl == kk), iif, float(N)))
        sel_ref[0] = sel.reshape(1, 1)

    call = pl.pallas_call(
        body,
        grid=(batch,),
        in_specs=[
            pl.BlockSpec((1, R, C), lambda i: (i, 0, 0)),
            pl.BlockSpec((1, R, C), lambda i: (i, 0, 0)),
        ],
        out_specs=[
            pl.BlockSpec((1, R, C), lambda i: (i, 0, 0)),
            pl.BlockSpec((1, 1, 1), lambda i: (i, 0, 0)),
        ],
        out_shape=[
            jax.ShapeDtypeStruct((batch, R, C), jnp.float32),
            jax.ShapeDtypeStruct((batch, 1, 1), jnp.float32),
        ],
    )

    def run(scores):
        pad = N - nvalid
        key = jax.random.key(42)
        g = jax.random.gumbel(key, (batch, nvalid), jnp.float32)
        sp = jnp.concatenate(
            [scores, jnp.full((batch, pad), -jnp.inf, jnp.float32)], axis=1
        ).reshape(batch, R, C)
        gp = jnp.concatenate(
            [g, jnp.zeros((batch, pad), jnp.float32)], axis=1
        ).reshape(batch, R, C)
        npv, sel = call(sp, gp)
        nucleus = npv.reshape(batch, N)[:, :nvalid]
        return sel.reshape(batch).astype(jnp.int32), nucleus

    return run


def kernel(scores):
    return _build(scores.shape[0], scores.shape[1], 10, 10)(scores)


# speedup vs baseline: 4.4015x; 1.0512x over previous
import functools

import jax
import jax.numpy as jnp
from jax import lax
from jax.experimental import pallas as pl

NUCLEUS_P = 0.9
TEMPERATURE = 1.0


def _roll(x, s, axis):
    n = x.shape[axis]
    s = s % n
    if s == 0:
        return x
    if axis == 0:
        return lax.concatenate([x[n - s:, :], x[: n - s, :]], 0)
    return lax.concatenate([x[:, n - s:], x[:, : n - s]], 1)


def _cumsum_last(x, log2n):
    n = x.shape[-1]
    for t in range(log2n):
        sh = 1 << t
        shifted = lax.concatenate(
            [jnp.zeros(x.shape[:-1] + (sh,), x.dtype), x[..., : n - sh]], x.ndim - 1)
        x = x + shifted
    return x


def _cumsum_ax0(x, log2n):
    n = x.shape[0]
    for t in range(log2n):
        sh = 1 << t
        shifted = lax.concatenate(
            [jnp.zeros((sh,) + x.shape[1:], x.dtype), x[: n - sh]], 0)
        x = x + shifted
    return x


@functools.lru_cache(maxsize=None)
def _build(batch, nvalid, log_r, log_c):
    R, C = 1 << log_r, 1 << log_c
    N = R * C
    LC, LN = log_c, log_r + log_c

    def body(s_ref, g_ref, np_ref, sel_ref):
        x = s_ref[0]
        iota0 = lax.broadcasted_iota(jnp.int32, (R, C), 0)
        iota1 = lax.broadcasted_iota(jnp.int32, (R, C), 1)

        def bit_of_i(bit):
            if bit >= LC:
                return (iota0 & (1 << (bit - LC))) != 0
            return (iota1 & (1 << bit)) != 0

        def compare_exchange(x, K, D):
            if D >= LC + 3:
                S = 1 << (D - LC)
                G = R // (2 * S)
                v = x.reshape(G, 2, S, C)
                a, b = v[:, 0], v[:, 1]
                mn = jnp.minimum(a, b)
                mx = jnp.maximum(a, b)
                gi = lax.broadcasted_iota(jnp.int32, (G, 1, 1), 0)
                asc = ((gi * (2 * S)) & (1 << (K - LC))) != 0
                na = jnp.where(asc, mn, mx)
                nb = jnp.where(asc, mx, mn)
                return lax.concatenate([na[:, None], nb[:, None]], 1).reshape(R, C)
            if D >= LC:
                axis, S = 0, 1 << (D - LC)
            else:
                axis, S = 1, 1 << D
            bD = bit_of_i(D)
            partner = jnp.where(bD, _roll(x, S, axis), _roll(x, -S, axis))
            mn = jnp.minimum(x, partner)
            mx = jnp.maximum(x, partner)
            return jnp.where(bD == bit_of_i(K), mx, mn)

        for K in range(1, LN + 1):
            for D in range(K - 1, -1, -1):
                x = compare_exchange(x, K, D)

        m = x[0, 0]
        e = jnp.exp((x - m) / TEMPERATURE)
        Z = jnp.sum(e)
        rz = 1.0 / Z
        q = e * rz

        row_sum = jnp.sum(q, axis=1, keepdims=True)
        row_incl = _cumsum_ax0(row_sum, log_r)
        row_excl = row_incl - row_sum
        rfull = row_incl <= NUCLEUS_P
        rstar = jnp.sum(jnp.where(rfull, 1.0, 0.0))
        iota0f = iota0.astype(jnp.float32)
        col_iota = lax.broadcasted_iota(jnp.int32, (R, 1), 0).astype(jnp.float32)
        is_rstar = iota0f == rstar
        brow = jnp.sum(jnp.where(is_rstar, q, 0.0), axis=0, keepdims=True)
        bexcl = jnp.sum(jnp.where(col_iota == rstar, row_excl, 0.0))
        bcs = _cumsum_last(brow, LC)
        bmask = (bexcl + bcs) <= NUCLEUS_P
        iif = iota0f * float(C) + iota1.astype(jnp.float32)
        mask = rfull | (is_rstar & bmask) | (iif == 0.0)
        Znuc = bexcl + jnp.max(jnp.where(bmask, bcs, 0.0))
        Znuc = jnp.maximum(Znuc, jnp.sum(jnp.where(iif == 0.0, q, 0.0)))
        npv = jnp.where(mask, q * (1.0 / Znuc), 0.0)
        np_ref[0] = npv

        g = g_ref[0]
        t = jnp.where(mask, g + x, -jnp.inf)
        tmax = jnp.max(t)
        jstar = jnp.min(jnp.where(t == tmax, iif, float(N)))
        qstar = jnp.max(jnp.where(iif == jstar, q, -1.0))

        so = s_ref[0]
        eo = jnp.exp((so - m) / TEMPERATURE)
        qo = eo * rz
        cgt = jnp.sum(jnp.where(qo > qstar, 1.0, 0.0))
        eqf = jnp.where(qo == qstar, 1.0, 0.0)
        kk = jstar - cgt
        ers = jnp.sum(eqf, axis=1, keepdims=True)
        erp = _cumsum_ax0(ers, log_r)
        eexcl = erp - ers
        rt = jnp.sum(jnp.where(erp <= kk, 1.0, 0.0))
        is_rt = iota0f == rt
        eqrow = jnp.sum(jnp.where(is_rt, eqf, 0.0), axis=0, keepdims=True)
        base = jnp.sum(jnp.where(col_iota == rt, eexcl, 0.0))
        ecs = _cumsum_last(eqrow, LC)
        excl = ecs - eqrow + base
        lane = lax.broadcasted_iota(jnp.int32, (1, C), 1).astype(jnp.float32)
        sel = jnp.min(jnp.where((eqrow > 0.0) & (excl == kk),
                                rt * float(C) + lane, float(N)))
        sel_ref[0] = sel.reshape(1, 1)

    call = pl.pallas_call(
        body,
        grid=(batch,),
        in_specs=[
            pl.BlockSpec((1, R, C), lambda i: (i, 0, 0)),
            pl.BlockSpec((1, R, C), lambda i: (i, 0, 0)),
        ],
        out_specs=[
            pl.BlockSpec((1, R, C), lambda i: (i, 0, 0)),
            pl.BlockSpec((1, 1, 1), lambda i: (i, 0, 0)),
        ],
        out_shape=[
            jax.ShapeDtypeStruct((batch, R, C), jnp.float32),
            jax.ShapeDtypeStruct((batch, 1, 1), jnp.float32),
        ],
    )

    def run(scores):
        pad = N - nvalid
        key = jax.random.key(42)
        g = jax.random.gumbel(key, (batch, nvalid), jnp.float32)
        sp = jnp.concatenate(
            [scores, jnp.full((batch, pad), -jnp.inf, jnp.float32)], axis=1
        ).reshape(batch, R, C)
        gp = jnp.concatenate(
            [g, jnp.zeros((batch, pad), jnp.float32)], axis=1
        ).reshape(batch, R, C)
        npv, sel = call(sp, gp)
        nucleus = npv.reshape(batch, N)[:, :nvalid]
        return sel.reshape(batch).astype(jnp.int32), nucleus

    return run


def kernel(scores):
    return _build(scores.shape[0], scores.shape[1], 10, 10)(scores)


# speedup vs baseline: 7.1550x; 1.6256x over previous
import functools

import jax
import jax.numpy as jnp
from jax import lax
from jax.experimental import pallas as pl

NUCLEUS_P = 0.9
TEMPERATURE = 1.0


def _roll(x, s, axis):
    n = x.shape[axis]
    s = s % n
    if s == 0:
        return x
    if axis == 0:
        return lax.concatenate([x[n - s:, :], x[: n - s, :]], 0)
    return lax.concatenate([x[:, n - s:], x[:, : n - s]], 1)


def _cumsum_last(x, log2n):
    n = x.shape[-1]
    for t in range(log2n):
        sh = 1 << t
        shifted = lax.concatenate(
            [jnp.zeros(x.shape[:-1] + (sh,), x.dtype), x[..., : n - sh]], x.ndim - 1)
        x = x + shifted
    return x


def _cumsum_ax0(x, log2n):
    n = x.shape[0]
    for t in range(log2n):
        sh = 1 << t
        shifted = lax.concatenate(
            [jnp.zeros((sh,) + x.shape[1:], x.dtype), x[: n - sh]], 0)
        x = x + shifted
    return x


@functools.lru_cache(maxsize=None)
def _build(batch, nvalid, log_r, log_c):
    R, C = 1 << log_r, 1 << log_c
    N = R * C
    LC, LN = log_c, log_r + log_c

    def body(s_ref, g_ref, np_ref, sel_ref):
        x = s_ref[0]
        iota0 = lax.broadcasted_iota(jnp.int32, (R, C), 0)
        iota1 = lax.broadcasted_iota(jnp.int32, (R, C), 1)

        def bit_of_i(bit):
            if bit >= LC:
                return (iota0 & (1 << (bit - LC))) != 0
            return (iota1 & (1 << bit)) != 0

        def compare_exchange(x, K, D):
            if D >= LC + 3:
                S = 1 << (D - LC)
                G = R // (2 * S)
                v = x.reshape(G, 2, S, C)
                a, b = v[:, 0], v[:, 1]
                mn = jnp.minimum(a, b)
                mx = jnp.maximum(a, b)
                gi = lax.broadcasted_iota(jnp.int32, (G, 1, 1), 0)
                asc = ((gi * (2 * S)) & (1 << (K - LC))) != 0
                na = jnp.where(asc, mn, mx)
                nb = jnp.where(asc, mx, mn)
                return lax.concatenate([na[:, None], nb[:, None]], 1).reshape(R, C)
            if D >= LC:
                axis, S = 0, 1 << (D - LC)
            else:
                axis, S = 1, 1 << D
            bD = bit_of_i(D)
            partner = jnp.where(bD, _roll(x, S, axis), _roll(x, -S, axis))
            mn = jnp.minimum(x, partner)
            mx = jnp.maximum(x, partner)
            return jnp.where(bD == bit_of_i(K), mx, mn)

        def ce_transposed(xT, K, D):
            S = 1 << D
            if S >= 8:
                G = C // (2 * S)
                v = xT.reshape(G, 2, S, R)
                a, b = v[:, 0], v[:, 1]
                mn = jnp.minimum(a, b)
                mx = jnp.maximum(a, b)
                if K >= LC:
                    ri = lax.broadcasted_iota(jnp.int32, (1, 1, R), 2)
                    asc = (ri & (1 << (K - LC))) != 0
                else:
                    gi = lax.broadcasted_iota(jnp.int32, (G, 1, 1), 0)
                    asc = ((gi * (2 * S)) & (1 << K)) != 0
                na = jnp.where(asc, mn, mx)
                nb = jnp.where(asc, mx, mn)
                return lax.concatenate([na[:, None], nb[:, None]], 1).reshape(C, R)
            i0 = lax.broadcasted_iota(jnp.int32, (C, R), 0)
            bD = (i0 & S) != 0
            if K >= LC:
                i1 = lax.broadcasted_iota(jnp.int32, (C, R), 1)
                asc = (i1 & (1 << (K - LC))) != 0
            else:
                asc = (i0 & (1 << K)) != 0
            partner = jnp.where(bD, _roll(xT, S, 0), _roll(xT, -S, 0))
            mn = jnp.minimum(xT, partner)
            mx = jnp.maximum(xT, partner)
            return jnp.where(bD == asc, mx, mn)

        x = x.T
        for K in range(1, LC + 1):
            for D in range(K - 1, -1, -1):
                x = ce_transposed(x, K, D)
        x = x.T
        for K in range(LC + 1, LN + 1):
            for D in range(K - 1, LC - 1, -1):
                x = compare_exchange(x, K, D)
            x = x.T
            for D in range(LC - 1, -1, -1):
                x = ce_transposed(x, K, D)
            x = x.T

        m = x[0, 0]
        e = jnp.exp((x - m) / TEMPERATURE)
        Z = jnp.sum(e)
        rz = 1.0 / Z
        q = e * rz

        row_sum = jnp.sum(q, axis=1, keepdims=True)
        row_incl = _cumsum_ax0(row_sum, log_r)
        row_excl = row_incl - row_sum
        rfull = row_incl <= NUCLEUS_P
        rstar = jnp.sum(jnp.where(rfull, 1.0, 0.0))
        iota0f = iota0.astype(jnp.float32)
        col_iota = lax.broadcasted_iota(jnp.int32, (R, 1), 0).astype(jnp.float32)
        is_rstar = iota0f == rstar
        brow = jnp.sum(jnp.where(is_rstar, q, 0.0), axis=0, keepdims=True)
        bexcl = jnp.sum(jnp.where(col_iota == rstar, row_excl, 0.0))
        bcs = _cumsum_last(brow, LC)
        bmask = (bexcl + bcs) <= NUCLEUS_P
        iif = iota0f * float(C) + iota1.astype(jnp.float32)
        mask = rfull | (is_rstar & bmask) | (iif == 0.0)
        Znuc = bexcl + jnp.max(jnp.where(bmask, bcs, 0.0))
        Znuc = jnp.maximum(Znuc, jnp.sum(jnp.where(iif == 0.0, q, 0.0)))
        npv = jnp.where(mask, q * (1.0 / Znuc), 0.0)
        np_ref[0] = npv

        g = g_ref[0]
        t = jnp.where(mask, g + x, -jnp.inf)
        tmax = jnp.max(t)
        jstar = jnp.min(jnp.where(t == tmax, iif, float(N)))
        qstar = jnp.max(jnp.where(iif == jstar, q, -1.0))

        so = s_ref[0]
        eo = jnp.exp((so - m) / TEMPERATURE)
        qo = eo * rz
        cgt = jnp.sum(jnp.where(qo > qstar, 1.0, 0.0))
        eqf = jnp.where(qo == qstar, 1.0, 0.0)
        kk = jstar - cgt
        ers = jnp.sum(eqf, axis=1, keepdims=True)
        erp = _cumsum_ax0(ers, log_r)
        eexcl = erp - ers
        rt = jnp.sum(jnp.where(erp <= kk, 1.0, 0.0))
        is_rt = iota0f == rt
        eqrow = jnp.sum(jnp.where(is_rt, eqf, 0.0), axis=0, keepdims=True)
        base = jnp.sum(jnp.where(col_iota == rt, eexcl, 0.0))
        ecs = _cumsum_last(eqrow, LC)
        excl = ecs - eqrow + base
        lane = lax.broadcasted_iota(jnp.int32, (1, C), 1).astype(jnp.float32)
        sel = jnp.min(jnp.where((eqrow > 0.0) & (excl == kk),
                                rt * float(C) + lane, float(N)))
        sel_ref[0] = sel.reshape(1, 1)

    call = pl.pallas_call(
        body,
        grid=(batch,),
        in_specs=[
            pl.BlockSpec((1, R, C), lambda i: (i, 0, 0)),
            pl.BlockSpec((1, R, C), lambda i: (i, 0, 0)),
        ],
        out_specs=[
            pl.BlockSpec((1, R, C), lambda i: (i, 0, 0)),
            pl.BlockSpec((1, 1, 1), lambda i: (i, 0, 0)),
        ],
        out_shape=[
            jax.ShapeDtypeStruct((batch, R, C), jnp.float32),
            jax.ShapeDtypeStruct((batch, 1, 1), jnp.float32),
        ],
    )

    def run(scores):
        pad = N - nvalid
        key = jax.random.key(42)
        g = jax.random.gumbel(key, (batch, nvalid), jnp.float32)
        sp = jnp.concatenate(
            [scores, jnp.full((batch, pad), -jnp.inf, jnp.float32)], axis=1
        ).reshape(batch, R, C)
        gp = jnp.concatenate(
            [g, jnp.zeros((batch, pad), jnp.float32)], axis=1
        ).reshape(batch, R, C)
        npv, sel = call(sp, gp)
        nucleus = npv.reshape(batch, N)[:, :nvalid]
        return sel.reshape(batch).astype(jnp.int32), nucleus

    return run


def kernel(scores):
    return _build(scores.shape[0], scores.shape[1], 10, 10)(scores)


# speedup vs baseline: 7.4989x; 1.0481x over previous
import functools

import jax
import jax.numpy as jnp
from jax import lax
from jax.experimental import pallas as pl

NUCLEUS_P = 0.9
TEMPERATURE = 1.0


def _cumsum_last(x, log2n):
    n = x.shape[-1]
    for t in range(log2n):
        sh = 1 << t
        shifted = lax.concatenate(
            [jnp.zeros(x.shape[:-1] + (sh,), x.dtype), x[..., : n - sh]], x.ndim - 1)
        x = x + shifted
    return x


def _cumsum_ax0(x, log2n):
    n = x.shape[0]
    for t in range(log2n):
        sh = 1 << t
        shifted = lax.concatenate(
            [jnp.zeros((sh,) + x.shape[1:], x.dtype), x[: n - sh]], 0)
        x = x + shifted
    return x


@functools.lru_cache(maxsize=None)
def _build(batch, nvalid, log_r, log_c):
    R, C = 1 << log_r, 1 << log_c
    N = R * C
    LC, LN = log_c, log_r + log_c

    def body(s_ref, g_ref, np_ref, sel_ref):
        x = s_ref[0]
        iota0 = lax.broadcasted_iota(jnp.int32, (R, C), 0)
        iota1 = lax.broadcasted_iota(jnp.int32, (R, C), 1)

        def compare_exchange(x, K, D):
            S = 1 << (D - LC)
            G = R // (2 * S)
            v = x.reshape(G, 2, S, C)
            a, b = v[:, 0], v[:, 1]
            mn = jnp.minimum(a, b)
            mx = jnp.maximum(a, b)
            gi = lax.broadcasted_iota(jnp.int32, (G, 1, 1), 0)
            asc = ((gi * (2 * S)) & (1 << (K - LC))) != 0
            na = jnp.where(asc, mn, mx)
            nb = jnp.where(asc, mx, mn)
            return lax.concatenate([na[:, None], nb[:, None]], 1).reshape(R, C)

        def ce_transposed(xT, K, D):
            S = 1 << D
            G = C // (2 * S)
            v = xT.reshape(G, 2, S, R)
            a, b = v[:, 0], v[:, 1]
            mn = jnp.minimum(a, b)
            mx = jnp.maximum(a, b)
            if K >= LC:
                ri = lax.broadcasted_iota(jnp.int32, (1, 1, R), 2)
                asc = (ri & (1 << (K - LC))) != 0
            else:
                gi = lax.broadcasted_iota(jnp.int32, (G, 1, 1), 0)
                asc = ((gi * (2 * S)) & (1 << K)) != 0
            na = jnp.where(asc, mn, mx)
            nb = jnp.where(asc, mx, mn)
            return lax.concatenate([na[:, None], nb[:, None]], 1).reshape(C, R)

        x = x.T
        for K in range(1, LC + 1):
            for D in range(K - 1, -1, -1):
                x = ce_transposed(x, K, D)
        x = x.T
        for K in range(LC + 1, LN + 1):
            for D in range(K - 1, LC - 1, -1):
                x = compare_exchange(x, K, D)
            x = x.T
            for D in range(LC - 1, -1, -1):
                x = ce_transposed(x, K, D)
            x = x.T

        m = x[0, 0]
        e = jnp.exp((x - m) / TEMPERATURE)
        Z = jnp.sum(e)
        rz = 1.0 / Z
        q = e * rz

        row_sum = jnp.sum(q, axis=1, keepdims=True)
        row_incl = _cumsum_ax0(row_sum, log_r)
        row_excl = row_incl - row_sum
        rfull = row_incl <= NUCLEUS_P
        rstar = jnp.sum(jnp.where(rfull, 1.0, 0.0))
        iota0f = iota0.astype(jnp.float32)
        col_iota = lax.broadcasted_iota(jnp.int32, (R, 1), 0).astype(jnp.float32)
        is_rstar = iota0f == rstar
        brow = jnp.sum(jnp.where(is_rstar, q, 0.0), axis=0, keepdims=True)
        bexcl = jnp.sum(jnp.where(col_iota == rstar, row_excl, 0.0))
        bcs = _cumsum_last(brow, LC)
        bmask = (bexcl + bcs) <= NUCLEUS_P
        iif = iota0f * float(C) + iota1.astype(jnp.float32)
        mask = rfull | (is_rstar & bmask) | (iif == 0.0)
        Znuc = bexcl + jnp.max(jnp.where(bmask, bcs, 0.0))
        Znuc = jnp.maximum(Znuc, jnp.sum(jnp.where(iif == 0.0, q, 0.0)))
        npv = jnp.where(mask, q * (1.0 / Znuc), 0.0)
        np_ref[0] = npv

        g = g_ref[0]
        t = jnp.where(mask, g + x, -jnp.inf)
        tmax = jnp.max(t)
        jstar = jnp.min(jnp.where(t == tmax, iif, float(N)))
        qstar = jnp.max(jnp.where(iif == jstar, q, -1.0))

        so = s_ref[0]
        eo = jnp.exp((so - m) / TEMPERATURE)
        qo = eo * rz
        cgt = jnp.sum(jnp.where(qo > qstar, 1.0, 0.0))
        eqf = jnp.where(qo == qstar, 1.0, 0.0)
        kk = jstar - cgt
        ers = jnp.sum(eqf, axis=1, keepdims=True)
        erp = _cumsum_ax0(ers, log_r)
        eexcl = erp - ers
        rt = jnp.sum(jnp.where(erp <= kk, 1.0, 0.0))
        is_rt = iota0f == rt
        eqrow = jnp.sum(jnp.where(is_rt, eqf, 0.0), axis=0, keepdims=True)
        base = jnp.sum(jnp.where(col_iota == rt, eexcl, 0.0))
        ecs = _cumsum_last(eqrow, LC)
        excl = ecs - eqrow + base
        lane = lax.broadcasted_iota(jnp.int32, (1, C), 1).astype(jnp.float32)
        sel = jnp.min(jnp.where((eqrow > 0.0) & (excl == kk),
                                rt * float(C) + lane, float(N)))
        sel_ref[0] = sel.reshape(1, 1)

    call = pl.pallas_call(
        body,
        grid=(batch,),
        in_specs=[
            pl.BlockSpec((1, R, C), lambda i: (i, 0, 0)),
            pl.BlockSpec((1, R, C), lambda i: (i, 0, 0)),
        ],
        out_specs=[
            pl.BlockSpec((1, R, C), lambda i: (i, 0, 0)),
            pl.BlockSpec((1, 1, 1), lambda i: (i, 0, 0)),
        ],
        out_shape=[
            jax.ShapeDtypeStruct((batch, R, C), jnp.float32),
            jax.ShapeDtypeStruct((batch, 1, 1), jnp.float32),
        ],
    )

    def run(scores):
        pad = N - nvalid
        key = jax.random.key(42)
        g = jax.random.gumbel(key, (batch, nvalid), jnp.float32)
        sp = jnp.concatenate(
            [scores, jnp.full((batch, pad), -jnp.inf, jnp.float32)], axis=1
        ).reshape(batch, R, C)
        gp = jnp.concatenate(
            [g, jnp.zeros((batch, pad), jnp.float32)], axis=1
        ).reshape(batch, R, C)
        npv, sel = call(sp, gp)
        nucleus = npv.reshape(batch, N)[:, :nvalid]
        return sel.reshape(batch).astype(jnp.int32), nucleus

    return run


def kernel(scores):
    return _build(scores.shape[0], scores.shape[1], 10, 10)(scores)
